# baseline (speedup 1.0000x reference)
import jax
import jax.numpy as jnp
from jax.experimental import pallas as pl
from jax.experimental.pallas import tpu as pltpu

_N = 10000
_K = 3
_NUM_LAYERS = 5
_EPS = 1e-5


def _gcn_norm(src, dst, w, n):
    deg = jnp.zeros((n,), w.dtype).at[dst].add(w)
    dis = jnp.where(deg > 0, jax.lax.rsqrt(jnp.where(deg > 0, deg, 1.0)), 0.0)
    return dis[src] * w * dis[dst]


def _tag_conv(h, src, dst, norm, W, b):
    out = h @ W[0]
    cur = h
    for k in range(1, _K + 1):
        cur = jnp.zeros((h.shape[0], cur.shape[1]), h.dtype).at[dst].add(norm[:, None] * cur[src])
        out = out + cur @ W[k]
    return out + b


def _bn(h, g, b):
    m = jnp.mean(h, axis=0)
    v = jnp.mean((h - m) ** 2, axis=0)
    return (h - m) * jax.lax.rsqrt(v + _EPS) * g + b


def _heads_kernel(emb_ref, wc_ref, bc_ref, wp_ref, bp_ref, cl_ref, pl_ref):
    emb = emb_ref[...]
    cl_ref[...] = emb @ wc_ref[...] + bc_ref[...]
    pl_ref[...] = emb @ wp_ref[...] + bp_ref[...]


def kernel(x, edge_index, edge_attr, W0, b0, Wmid, bmid, gammas, betas, Wc, bc, Wp, bp):
    src, dst = edge_index[0], edge_index[1]
    ew = edge_attr.squeeze()
    norm = _gcn_norm(src, dst, ew, x.shape[0])
    h = _tag_conv(x, src, dst, norm, W0, b0)
    h = _bn(h, gammas[0], betas[0])
    h = jnp.where(h > 0, h, 0.2 * h)
    for i in range(1, _NUM_LAYERS - 1):
        h = _tag_conv(h, src, dst, norm, Wmid[i - 1], bmid[i - 1])
        h = _bn(h, gammas[i], betas[i])
        h = jnp.where(h > 0, h, 0.2 * h)
    emb = _tag_conv(h, src, dst, norm, Wmid[_NUM_LAYERS - 2], bmid[_NUM_LAYERS - 2])

    nch = Wc.shape[1]
    npw = Wp.shape[1]
    cl, plog = pl.pallas_call(
        _heads_kernel,
        out_shape=(
            jax.ShapeDtypeStruct((_N, nch), jnp.float32),
            jax.ShapeDtypeStruct((_N, npw), jnp.float32),
        ),
    )(emb, Wc, bc, Wp, bp)
    return (cl, plog)


# speedup vs baseline: 2.5026x; 2.5026x over previous
import functools

import jax
import jax.numpy as jnp
from jax import lax
from jax.experimental import pallas as pl
from jax.experimental.pallas import tpu as pltpu
from jax.experimental.pallas import tpu_sc as plsc

_N = 10000
_NP = 10240
_E = 320000
_EP = 327680
_HID = 256
_K = 3
_LAYERS = 5
_EPS = 1e-5

_PW = _EP // 16
_NW = _EP // 32
_CHUNK = 128
_NCHUNK = _PW // _CHUNK
_ROWS_W = _NP // 16
_BLK = _NP // 8

_mesh = plsc.VectorSubcoreMesh(core_axis_name="c", subcore_axis_name="s")


def _zero16():
    return jnp.zeros((16,), jnp.float32)


def _dis_body(parts_ref, dis_ref):
    deg = parts_ref[0, :, 0]
    safe = jnp.where(deg > 0, deg, 1.0)
    dis_ref[...] = jnp.where(deg > 0, lax.rsqrt(safe), 0.0)


def _norm_body(srcp, dstp, wp, dis, nrm, src_v, dst_v, w_v, dis_v, out_v):
    c = lax.axis_index("c")
    s = lax.axis_index("s")
    wid = s * 2 + c
    base = wid * _NW
    pltpu.sync_copy(srcp.at[pl.ds(base, _NW)], src_v)
    pltpu.sync_copy(dstp.at[pl.ds(base, _NW)], dst_v)
    pltpu.sync_copy(wp.at[pl.ds(base, _NW)], w_v)
    pltpu.sync_copy(dis, dis_v)

    def gb(g, carry):
        sl = pl.ds(g * 16, 16)
        s16 = src_v[sl]
        d16 = dst_v[sl]
        w16 = w_v[sl]
        a = plsc.load_gather(dis_v, [s16])
        b = plsc.load_gather(dis_v, [d16])
        out_v[sl] = a * w16 * b
        return carry

    lax.fori_loop(0, _NW // 16, gb, None)
    pltpu.sync_copy(out_v, nrm.at[pl.ds(base, _NW)])


@functools.partial(
    pl.kernel,
    out_type=jax.ShapeDtypeStruct((_EP,), jnp.float32),
    mesh=_mesh,
    compiler_params=pltpu.CompilerParams(needs_layout_passes=False),
    scratch_types=[
        pltpu.VMEM((_NW,), jnp.int32),
        pltpu.VMEM((_NW,), jnp.int32),
        pltpu.VMEM((_NW,), jnp.float32),
        pltpu.VMEM((_NP,), jnp.float32),
        pltpu.VMEM((_NW,), jnp.float32),
    ],
)
def _norm_kernel(srcp, dstp, wp, dis, nrm, src_v, dst_v, w_v, dis_v, out_v):
    _norm_body(srcp, dstp, wp, dis, nrm, src_v, dst_v, w_v, dis_v, out_v)


_PCH = 128
_SEG = 4096
_SCH = _SEG // _PCH
_NSEG = _PW // _SEG


def _prop_body(h, srcp, dst2, nrmp, out, src_v, dst_v, nrm_v, rows, acc):
    c = lax.axis_index("c")
    s = lax.axis_index("s")

    def zb(i, carry):
        for f in range(8):
            rows[i, pl.ds(f * 16, 16)] = _zero16()
        return carry

    lax.fori_loop(0, _PCH, zb, None)
    for t in range(_ROWS_W // _PCH):
        pltpu.sync_copy(rows, acc.at[pl.ds(s * _ROWS_W + t * _PCH, _PCH)])
    plsc.subcore_barrier()

    def seg(j, carry):
        base = s * _PW + j * _SEG
        pltpu.sync_copy(srcp.at[pl.ds(base, _SEG)], src_v)
        pltpu.sync_copy(nrmp.at[pl.ds(base, _SEG)], nrm_v)
        pltpu.sync_copy(dst2.at[pl.ds(s * (_NSEG * _SCH) + j * _SCH, _SCH)], dst_v)

        def chunk(k, carry1):
            pltpu.sync_copy(h.at[c].at[src_v.at[pl.ds(k * _PCH, _PCH)]], rows)

            def grp(g, carry2):
                nv = nrm_v[pl.ds(k * _PCH + g * 16, 16)]
                for l in range(16):
                    scal = lax.squeeze(lax.slice(nv, (l,), (l + 1,)), (0,))
                    spl = lax.broadcast_in_dim(scal, (16,), ())
                    e = g * 16 + l
                    for f in range(8):
                        sl = pl.ds(f * 16, 16)
                        rows[e, sl] = rows[e, sl] * spl
                return carry2

            lax.fori_loop(0, _PCH // 16, grp, None)
            pltpu.sync_copy(rows, acc.at[dst_v.at[k]], add=True)
            return carry1

        lax.fori_loop(0, _SCH, chunk, None)
        return carry

    lax.fori_loop(0, _NSEG, seg, None)
    plsc.subcore_barrier()
    pltpu.sync_copy(
        acc.at[pl.ds(s * _ROWS_W, _ROWS_W)], out.at[c, pl.ds(s * _ROWS_W, _ROWS_W)]
    )


@functools.partial(
    pl.kernel,
    out_type=jax.ShapeDtypeStruct((2, _NP, 128), jnp.float32),
    mesh=_mesh,
    compiler_params=pltpu.CompilerParams(needs_layout_passes=False),
    scratch_types=[
        pltpu.VMEM((_SEG,), jnp.int32),
        pltpu.VMEM((_SCH, _PCH), jnp.int32),
        pltpu.VMEM((_SEG,), jnp.float32),
        pltpu.VMEM((_PCH, 128), jnp.float32),
        pltpu.VMEM_SHARED((_NP, 128), jnp.float32),
    ],
)
def _prop_kernel(h, srcp, dst2, nrmp, out, src_v, dst_v, nrm_v, rows, acc):
    _prop_body(h, srcp, dst2, nrmp, out, src_v, dst_v, nrm_v, rows, acc)


def _mm_body(y0, y1, y2, y3, w_ref, b_ref, raw_ref, st_ref):
    i = pl.program_id(0)
    ys = (y0, y1, y2, y3)
    gid = lax.broadcasted_iota(jnp.int32, (_BLK, 1), 0) + i * _BLK
    mask = (gid < _N).astype(jnp.float32)

    @pl.when(i == 0)
    def _():
        st_ref[...] = jnp.zeros_like(st_ref)

    for cout in range(2):
        acc = jnp.broadcast_to(b_ref[cout], (_BLK, 128))
        for k in range(4):
            for cin in range(2):
                acc = acc + jnp.dot(
                    ys[k][cin], w_ref[k, cin, cout],
                    preferred_element_type=jnp.float32,
                    precision=lax.Precision.HIGHEST,
                )
        raw_ref[cout] = acc
        m = acc * mask
        st_ref[0, cout] = st_ref[0, cout] + jnp.sum(m, axis=0)
        st_ref[1, cout] = st_ref[1, cout] + jnp.sum(m * acc, axis=0)


def _mm_call(y0, y1, y2, y3, w, b):
    yspec = pl.BlockSpec((2, _BLK, 128), lambda i: (0, i, 0))
    return pl.pallas_call(
        _mm_body,
        grid=(_NP // _BLK,),
        in_specs=[
            yspec, yspec, yspec, yspec,
            pl.BlockSpec((4, 2, 2, 128, 128), lambda i: (0, 0, 0, 0, 0)),
            pl.BlockSpec((2, 128), lambda i: (0, 0)),
        ],
        out_specs=(
            pl.BlockSpec((2, _BLK, 128), lambda i: (0, i, 0)),
            pl.BlockSpec((2, 2, 128), lambda i: (0, 0, 0)),
        ),
        out_shape=(
            jax.ShapeDtypeStruct((2, _NP, 128), jnp.float32),
            jax.ShapeDtypeStruct((2, 2, 128), jnp.float32),
        ),
    )(y0, y1, y2, y3, w, b)


def _bn_body(raw_ref, st_ref, g_ref, b_ref, out_ref):
    inv_n = 1.0 / _N
    for c in range(2):
        mu = st_ref[0, c] * inv_n
        var = st_ref[1, c] * inv_n - mu * mu
        scale = lax.rsqrt(var + _EPS) * g_ref[c]
        y = (raw_ref[c] - mu[None, :]) * scale[None, :] + b_ref[c][None, :]
        out_ref[c] = jnp.where(y > 0, y, 0.2 * y)


def _bn_call(raw, st, g, b):
    return pl.pallas_call(
        _bn_body,
        grid=(_NP // _BLK,),
        in_specs=[
            pl.BlockSpec((2, _BLK, 128), lambda i: (0, i, 0)),
            pl.BlockSpec((2, 2, 128), lambda i: (0, 0, 0)),
            pl.BlockSpec((2, 128), lambda i: (0, 0)),
            pl.BlockSpec((2, 128), lambda i: (0, 0)),
        ],
        out_specs=pl.BlockSpec((2, _BLK, 128), lambda i: (0, i, 0)),
        out_shape=jax.ShapeDtypeStruct((2, _NP, 128), jnp.float32),
    )(raw, st, g, b)


def _heads_body(emb_ref, wc_ref, bc_ref, wp_ref, bp_ref, cl_ref, pl_ref):
    e0 = emb_ref[0]
    e1 = emb_ref[1]
    cl_ref[...] = (
        jnp.dot(e0, wc_ref[0], preferred_element_type=jnp.float32, precision=lax.Precision.HIGHEST)
        + jnp.dot(e1, wc_ref[1], preferred_element_type=jnp.float32, precision=lax.Precision.HIGHEST)
        + bc_ref[...]
    )
    pl_ref[...] = (
        jnp.dot(e0, wp_ref[0], preferred_element_type=jnp.float32, precision=lax.Precision.HIGHEST)
        + jnp.dot(e1, wp_ref[1], preferred_element_type=jnp.float32, precision=lax.Precision.HIGHEST)
        + bp_ref[...]
    )


def _heads_call(emb, wc, bc, wp, bp):
    nch, npw = wc.shape[-1], wp.shape[-1]
    return pl.pallas_call(
        _heads_body,
        grid=(_NP // _BLK,),
        in_specs=[
            pl.BlockSpec((2, _BLK, 128), lambda i: (0, i, 0)),
            pl.BlockSpec((2, 128, nch), lambda i: (0, 0, 0)),
            pl.BlockSpec((1, nch), lambda i: (0, 0)),
            pl.BlockSpec((2, 128, npw), lambda i: (0, 0, 0)),
            pl.BlockSpec((1, npw), lambda i: (0, 0)),
        ],
        out_specs=(
            pl.BlockSpec((_BLK, nch), lambda i: (i, 0)),
            pl.BlockSpec((_BLK, npw), lambda i: (i, 0)),
        ),
        out_shape=(
            jax.ShapeDtypeStruct((_NP, nch), jnp.float32),
            jax.ShapeDtypeStruct((_NP, npw), jnp.float32),
        ),
    )(emb, wc, bc, wp, bp)


def kernel(x, edge_index, edge_attr, W0, b0, Wmid, bmid, gammas, betas, Wc, bc, Wp, bp):
    src = jnp.pad(edge_index[0], (0, _EP - _E))
    dst = jnp.pad(edge_index[1], (0, _EP - _E))
    w = jnp.pad(edge_attr[:, 0], (0, _EP - _E))
    dst2 = dst.reshape(_EP // _PCH, _PCH)

    ones_h = jnp.ones((2, _NP, 128), jnp.float32)
    parts = _prop_kernel(ones_h, src, dst2, w)
    dis = pl.pallas_call(
        _dis_body, out_shape=jax.ShapeDtypeStruct((_NP,), jnp.float32)
    )(parts)
    nrm = _norm_kernel(src, dst, w, dis)

    xp = jnp.pad(x, ((0, _NP - _N), (0, _HID - x.shape[1])))
    h = xp.reshape(_NP, 2, 128).transpose(1, 0, 2)

    w0p = jnp.pad(W0, ((0, 0), (0, _HID - W0.shape[1]), (0, 0)))
    wall = jnp.concatenate([w0p[None], Wmid], axis=0)
    wr = wall.reshape(_LAYERS, 4, 2, 128, 2, 128).transpose(0, 1, 2, 4, 3, 5)
    ball = jnp.concatenate([b0[None], bmid], axis=0).reshape(_LAYERS, 2, 128)
    g2 = gammas.reshape(_LAYERS - 1, 2, 128)
    be2 = betas.reshape(_LAYERS - 1, 2, 128)

    for i in range(_LAYERS):
        y1 = _prop_kernel(h, src, dst2, nrm)
        y2 = _prop_kernel(y1, src, dst2, nrm)
        y3 = _prop_kernel(y2, src, dst2, nrm)
        raw, st = _mm_call(h, y1, y2, y3, wr[i], ball[i])
        if i < _LAYERS - 1:
            h = _bn_call(raw, st, g2[i], be2[i])
        else:
            emb = raw

    wc2 = Wc.reshape(2, 128, Wc.shape[1])
    wp2 = Wp.reshape(2, 128, Wp.shape[1])
    cl, plog = _heads_call(emb, wc2, bc[None], wp2, bp[None])
    return (cl[:_N], plog[:_N])
